# baseline (speedup 1.0000x reference)
import jax
import jax.numpy as jnp
from jax.experimental import pallas as pl

N_NODES = 10000
N_GRAPHS = 64
N_LAYERS = 3
HID = 128


def _head_body(rp_ref, rl_ref, w1_ref, w2_ref, o_ref):
    s = rp_ref[...] + rl_ref[...]
    h = jnp.maximum(jnp.dot(s, w1_ref[...], preferred_element_type=jnp.float32), 0.0)
    o_ref[...] = jnp.dot(h, w2_ref[...], preferred_element_type=jnp.float32)


def _head(rp, rl, W_d1, W_d2):
    w2p = jnp.pad(W_d2, ((0, 0), (0, 127)))
    out = pl.pallas_call(
        _head_body,
        out_shape=jax.ShapeDtypeStruct((N_GRAPHS, 128), jnp.float32),
    )(rp, rl, W_d1, w2p)
    return out[:, :1]


def _gcn_layer(h, e_ij, src, dst, W, n_nodes):
    m = jax.nn.relu(h[src] @ W + e_ij)
    agg = jax.ops.segment_sum(m, dst, num_segments=n_nodes)
    return jax.nn.relu(h + agg)


def _gat_layer(h, e_ij, src, dst, W, A, n_nodes):
    z = h @ W
    logits = z[src] @ A[0] + z[dst] @ A[1] + e_ij @ A[2]
    logits = jax.nn.leaky_relu(logits, 0.2)
    mx = jax.ops.segment_max(logits, dst, num_segments=n_nodes)
    mx = jnp.where(jnp.isfinite(mx), mx, 0.0)
    ex = jnp.exp(logits - mx[dst])
    denom = jax.ops.segment_sum(ex, dst, num_segments=n_nodes)
    alpha = ex / (denom[dst] + 1e-9)
    agg = jax.ops.segment_sum(alpha[:, None] * z[src], dst, num_segments=n_nodes)
    return jax.nn.relu(h + agg)


def kernel(x_p, e_p, edge_index_p, graph_id_p, x_l, e_l, edge_index_l, graph_id_l,
           W_node_p, W_edge_p, W_node_l, W_edge_l, W_gcn, W_gat, A_gat, W_d1, W_d2):
    h_p = x_p @ W_node_p
    eij_p = e_p @ W_edge_p
    h_l = x_l @ W_node_l
    eij_l = e_l @ W_edge_l
    src_p, dst_p = edge_index_p[0], edge_index_p[1]
    src_l, dst_l = edge_index_l[0], edge_index_l[1]
    for i in range(N_LAYERS):
        h_p = _gcn_layer(h_p, eij_p, src_p, dst_p, W_gcn[i], N_NODES)
    readout_p = jax.ops.segment_sum(h_p, graph_id_p, num_segments=N_GRAPHS)
    for j in range(N_LAYERS):
        h_l = _gat_layer(h_l, eij_l, src_l, dst_l, W_gat[j], A_gat[j], N_NODES)
    readout_l = jax.ops.segment_sum(h_l, graph_id_l, num_segments=N_GRAPHS)
    return _head(readout_p, readout_l, W_d1, W_d2)


# speedup vs baseline: 2.7484x; 2.7484x over previous
import functools

import jax
import jax.numpy as jnp
from jax import lax
from jax.experimental import pallas as pl
from jax.experimental.pallas import tpu as pltpu
from jax.experimental.pallas import tpu_sc as plsc

V = 10000
E = 320000
HID = 128
NG = 64
NL = 3

NC = 2
NS = 16
NW = NC * NS
EPW = 10240
E_PAD = NW * EPW
C = 128
NCHUNK = EPW // C
VACC = 10240

_f32 = jnp.float32
_i32 = jnp.int32

_SC_MESH = plsc.VectorSubcoreMesh(core_axis_name="c", subcore_axis_name="s")


def _mm_body(x_ref, w_ref, o_ref):
    o_ref[...] = jnp.dot(x_ref[...], w_ref[...], preferred_element_type=_f32)


def _mm(x, w, bm):
    m, k = x.shape
    n = w.shape[1]
    return pl.pallas_call(
        _mm_body,
        grid=(m // bm,),
        in_specs=[
            pl.BlockSpec((bm, k), lambda i: (i, 0)),
            pl.BlockSpec((k, n), lambda i: (0, 0)),
        ],
        out_specs=pl.BlockSpec((bm, n), lambda i: (i, 0)),
        out_shape=jax.ShapeDtypeStruct((m, n), _f32),
    )(x, w)


def _gcn_update_body(h_ref, p0_ref, p1_ref, w_ref, hn_ref, hw_ref):
    hn = jnp.maximum(h_ref[...] + p0_ref[...] + p1_ref[...], 0.0)
    hn_ref[...] = hn
    hw_ref[...] = jnp.dot(hn, w_ref[...], preferred_element_type=_f32)


def _gcn_update(h, p0, p1, w, bm=2000):
    return pl.pallas_call(
        _gcn_update_body,
        grid=(V // bm,),
        in_specs=[
            pl.BlockSpec((bm, HID), lambda i: (i, 0)),
            pl.BlockSpec((bm, HID), lambda i: (i, 0)),
            pl.BlockSpec((bm, HID), lambda i: (i, 0)),
            pl.BlockSpec((HID, HID), lambda i: (0, 0)),
        ],
        out_specs=[
            pl.BlockSpec((bm, HID), lambda i: (i, 0)),
            pl.BlockSpec((bm, HID), lambda i: (i, 0)),
        ],
        out_shape=[
            jax.ShapeDtypeStruct((V, HID), _f32),
            jax.ShapeDtypeStruct((V, HID), _f32),
        ],
    )(h, p0, p1, w)


def _gat_pre_first_body(h_ref, w_ref, a_ref, z_ref, s_ref):
    z = jnp.dot(h_ref[...], w_ref[...], preferred_element_type=_f32)
    z_ref[...] = z
    s_ref[...] = jnp.dot(z, a_ref[...], preferred_element_type=_f32)


def _gat_pre_first(h, w, a01, bm=2000):
    return pl.pallas_call(
        _gat_pre_first_body,
        grid=(V // bm,),
        in_specs=[
            pl.BlockSpec((bm, HID), lambda i: (i, 0)),
            pl.BlockSpec((HID, HID), lambda i: (0, 0)),
            pl.BlockSpec((HID, HID), lambda i: (0, 0)),
        ],
        out_specs=[
            pl.BlockSpec((bm, HID), lambda i: (i, 0)),
            pl.BlockSpec((bm, HID), lambda i: (i, 0)),
        ],
        out_shape=[
            jax.ShapeDtypeStruct((V, HID), _f32),
            jax.ShapeDtypeStruct((V, HID), _f32),
        ],
    )(h, w, a01)


def _gat_pre_body(h_ref, p0_ref, p1_ref, w_ref, a_ref, hn_ref, z_ref, s_ref):
    hn = jnp.maximum(h_ref[...] + p0_ref[...] + p1_ref[...], 0.0)
    hn_ref[...] = hn
    z = jnp.dot(hn, w_ref[...], preferred_element_type=_f32)
    z_ref[...] = z
    s_ref[...] = jnp.dot(z, a_ref[...], preferred_element_type=_f32)


def _gat_pre(h, p0, p1, w, a01, bm=2000):
    return pl.pallas_call(
        _gat_pre_body,
        grid=(V // bm,),
        in_specs=[
            pl.BlockSpec((bm, HID), lambda i: (i, 0)),
            pl.BlockSpec((bm, HID), lambda i: (i, 0)),
            pl.BlockSpec((bm, HID), lambda i: (i, 0)),
            pl.BlockSpec((HID, HID), lambda i: (0, 0)),
            pl.BlockSpec((HID, HID), lambda i: (0, 0)),
        ],
        out_specs=[
            pl.BlockSpec((bm, HID), lambda i: (i, 0)),
            pl.BlockSpec((bm, HID), lambda i: (i, 0)),
            pl.BlockSpec((bm, HID), lambda i: (i, 0)),
        ],
        out_shape=[
            jax.ShapeDtypeStruct((V, HID), _f32),
            jax.ShapeDtypeStruct((V, HID), _f32),
            jax.ShapeDtypeStruct((V, HID), _f32),
        ],
    )(h, p0, p1, w, a01)


def _gat_t_body(e_ref, we_ref, a2_ref, t_ref):
    w2 = jnp.dot(we_ref[...], a2_ref[...], preferred_element_type=_f32)
    t_ref[...] = jnp.dot(e_ref[...], w2, preferred_element_type=_f32)


def _gat_t(e_pad, we_pad, a2col, bm=2048):
    m = e_pad.shape[0]
    return pl.pallas_call(
        _gat_t_body,
        grid=(m // bm,),
        in_specs=[
            pl.BlockSpec((bm, 8), lambda i: (i, 0)),
            pl.BlockSpec((8, HID), lambda i: (0, 0)),
            pl.BlockSpec((HID, 8), lambda i: (0, 0)),
        ],
        out_specs=pl.BlockSpec((bm, 8), lambda i: (i, 0)),
        out_shape=jax.ShapeDtypeStruct((m, 8), _f32),
    )(e_pad, we_pad, a2col)


def _readout_body(h_ref, p0_ref, p1_ref, gid_ref, o_ref):
    hf = jnp.maximum(h_ref[...] + p0_ref[...] + p1_ref[...], 0.0)
    iota = lax.broadcasted_iota(_i32, (NG, V), 0).astype(_f32)
    onehot = (gid_ref[...] == iota).astype(_f32)
    o_ref[...] = jnp.dot(onehot, hf, preferred_element_type=_f32)


def _readout(h, p0, p1, gid_row):
    return pl.pallas_call(
        _readout_body,
        out_shape=jax.ShapeDtypeStruct((NG, HID), _f32),
    )(h, p0, p1, gid_row)


def _head_body(rp_ref, rl_ref, w1_ref, w2_ref, o_ref):
    s = rp_ref[...] + rl_ref[...]
    hdn = jnp.maximum(jnp.dot(s, w1_ref[...], preferred_element_type=_f32), 0.0)
    o_ref[...] = jnp.dot(hdn, w2_ref[...], preferred_element_type=_f32)


def _head(rp, rl, w1, w2pad):
    return pl.pallas_call(
        _head_body,
        out_shape=jax.ShapeDtypeStruct((NG, 8), _f32),
    )(rp, rl, w1, w2pad)


def _zero_rows(zb):
    def zrow(r, carry):
        for c8 in range(8):
            zb[r, pl.ds(c8 * 16, 16)] = jnp.zeros((16,), _f32)
        return carry
    lax.fori_loop(0, 128, zrow, 0)


def _gcn_edge_body(hw, eij, src, dst, out, srcb, dstb, rows, eijb, acc, sem):
    cid = lax.axis_index("c")
    sid = lax.axis_index("s")
    wid = sid * NC + cid
    _zero_rows(rows)
    for kk in range(VACC // NS // 128):
        pltpu.sync_copy(rows, acc.at[pl.ds(sid * (VACC // NS) + kk * 128, 128), :])
    plsc.subcore_barrier()
    base = wid * EPW

    def chunk(k, carry):
        off = base + k * C
        pltpu.sync_copy(src.at[pl.ds(off, C)], srcb)
        pltpu.sync_copy(dst.at[pl.ds(off, C)], dstb)
        pltpu.sync_copy(eij.at[pl.ds(off, C), :], eijb)
        pltpu.async_copy(hw.at[srcb], rows, sem).wait()

        def row(r, rc):
            for c8 in range(8):
                s16 = pl.ds(c8 * 16, 16)
                rows[r, s16] = jnp.maximum(rows[r, s16] + eijb[r, s16], 0.0)
            return rc
        lax.fori_loop(0, C, row, 0, unroll=2)
        pltpu.sync_copy(rows, acc.at[dstb], add=True)
        return carry
    lax.fori_loop(0, NCHUNK, chunk, 0)
    plsc.subcore_barrier()
    for kk in range(5):
        r0 = sid * 640 + kk * 128
        pltpu.sync_copy(acc.at[pl.ds(r0, 128), :], out.at[cid, pl.ds(r0, 128), :])


@functools.partial(
    pl.kernel,
    out_type=jax.ShapeDtypeStruct((NC, VACC, HID), _f32),
    mesh=_SC_MESH,
    compiler_params=pltpu.CompilerParams(needs_layout_passes=False, has_side_effects=True),
    scratch_types=[
        pltpu.VMEM((C,), _i32),
        pltpu.VMEM((C,), _i32),
        pltpu.VMEM((C, HID), _f32),
        pltpu.VMEM((C, HID), _f32),
        pltpu.VMEM_SHARED((VACC, HID), _f32),
        pltpu.SemaphoreType.DMA,
    ],
)
def _gcn_edge(hw, eij, src, dst, out, srcb, dstb, rows, eijb, acc, sem):
    _gcn_edge_body(hw, eij, src, dst, out, srcb, dstb, rows, eijb, acc, sem)


def _gat_scalar_body(s0, s1, t, src, dst, ex_out, d_out,
                     s0b, s1b, tb, srcb, dstb, exb, zb1, dacc):
    cid = lax.axis_index("c")
    sid = lax.axis_index("s")
    wid = sid * NC + cid
    def zrow(r, carry):
        zb1[pl.ds(r * 16, 16)] = jnp.zeros((16,), _f32)
        return carry
    lax.fori_loop(0, VACC // NS // 16, zrow, 0)
    pltpu.sync_copy(zb1, dacc.at[pl.ds(sid * (VACC // NS), VACC // NS)])
    pltpu.sync_copy(s0, s0b)
    pltpu.sync_copy(s1, s1b)
    plsc.subcore_barrier()
    base = wid * EPW

    def chunk(k, carry):
        off = base + k * C
        pltpu.sync_copy(src.at[pl.ds(off, C)], srcb)
        pltpu.sync_copy(dst.at[pl.ds(off, C)], dstb)
        pltpu.sync_copy(t.at[pl.ds(off, C)], tb)
        for g in range(8):
            s16 = pl.ds(g * 16, 16)
            lg = (plsc.load_gather(s0b, [srcb[s16]])
                  + plsc.load_gather(s1b, [dstb[s16]])
                  + tb[s16])
            lg = jnp.maximum(lg, 0.2 * lg)
            exb[s16] = jnp.exp(lg)
        pltpu.sync_copy(exb, ex_out.at[pl.ds(off, C)])
        pltpu.sync_copy(exb, dacc.at[dstb], add=True)
        return carry
    lax.fori_loop(0, NCHUNK, chunk, 0)
    plsc.subcore_barrier()
    sl = pl.ds(sid * (VACC // NS), VACC // NS)
    pltpu.sync_copy(dacc.at[sl], d_out.at[cid, sl])


@functools.partial(
    pl.kernel,
    out_type=[
        jax.ShapeDtypeStruct((E_PAD,), _f32),
        jax.ShapeDtypeStruct((NC, VACC), _f32),
    ],
    mesh=_SC_MESH,
    compiler_params=pltpu.CompilerParams(needs_layout_passes=False, has_side_effects=True),
    scratch_types=[
        pltpu.VMEM((VACC,), _f32),
        pltpu.VMEM((VACC,), _f32),
        pltpu.VMEM((C,), _f32),
        pltpu.VMEM((C,), _i32),
        pltpu.VMEM((C,), _i32),
        pltpu.VMEM((C,), _f32),
        pltpu.VMEM((VACC // NS,), _f32),
        pltpu.VMEM_SHARED((VACC,), _f32),
    ],
)
def _gat_scalar(s0, s1, t, src, dst, ex_out, d_out,
                s0b, s1b, tb, srcb, dstb, exb, zb1, dacc):
    _gat_scalar_body(s0, s1, t, src, dst, ex_out, d_out,
                     s0b, s1b, tb, srcb, dstb, exb, zb1, dacc)


def _gat_rows_body(z, ex, d, src, dst, out,
                   srcb, dstb, exb, ab, rows, dbuf, d2buf, acc, sem):
    cid = lax.axis_index("c")
    sid = lax.axis_index("s")
    wid = sid * NC + cid
    _zero_rows(rows)
    for kk in range(VACC // NS // 128):
        pltpu.sync_copy(rows, acc.at[pl.ds(sid * (VACC // NS) + kk * 128, 128), :])
    pltpu.sync_copy(d.at[0], dbuf)
    pltpu.sync_copy(d.at[1], d2buf)

    def comb(i, carry):
        s16 = pl.ds(i * 16, 16)
        dbuf[s16] = 1.0 / (dbuf[s16] + d2buf[s16] + 1e-9)
        return carry
    lax.fori_loop(0, VACC // 16, comb, 0, unroll=4)
    plsc.subcore_barrier()
    base = wid * EPW

    def chunk(k, carry):
        off = base + k * C
        pltpu.sync_copy(src.at[pl.ds(off, C)], srcb)
        pltpu.sync_copy(dst.at[pl.ds(off, C)], dstb)
        pltpu.sync_copy(ex.at[pl.ds(off, C)], exb)
        pltpu.async_copy(z.at[srcb], rows, sem).wait()
        for g in range(8):
            s16 = pl.ds(g * 16, 16)
            ab[s16] = exb[s16] * plsc.load_gather(dbuf, [dstb[s16]])

        def row(r, rc):
            asp = plsc.load_gather(ab, [jnp.zeros((16,), _i32) + r])
            for c8 in range(8):
                s16 = pl.ds(c8 * 16, 16)
                rows[r, s16] = rows[r, s16] * asp
            return rc
        lax.fori_loop(0, C, row, 0, unroll=2)
        pltpu.sync_copy(rows, acc.at[dstb], add=True)
        return carry
    lax.fori_loop(0, NCHUNK, chunk, 0)
    plsc.subcore_barrier()
    for kk in range(5):
        r0 = sid * 640 + kk * 128
        pltpu.sync_copy(acc.at[pl.ds(r0, 128), :], out.at[cid, pl.ds(r0, 128), :])


@functools.partial(
    pl.kernel,
    out_type=jax.ShapeDtypeStruct((NC, VACC, HID), _f32),
    mesh=_SC_MESH,
    compiler_params=pltpu.CompilerParams(needs_layout_passes=False, has_side_effects=True),
    scratch_types=[
        pltpu.VMEM((C,), _i32),
        pltpu.VMEM((C,), _i32),
        pltpu.VMEM((C,), _f32),
        pltpu.VMEM((C,), _f32),
        pltpu.VMEM((C, HID), _f32),
        pltpu.VMEM((VACC,), _f32),
        pltpu.VMEM((VACC,), _f32),
        pltpu.VMEM_SHARED((VACC, HID), _f32),
        pltpu.SemaphoreType.DMA,
    ],
)
def _gat_rows(z, ex, d, src, dst, out,
              srcb, dstb, exb, ab, rows, dbuf, d2buf, acc, sem):
    _gat_rows_body(z, ex, d, src, dst, out,
                   srcb, dstb, exb, ab, rows, dbuf, d2buf, acc, sem)


def _pad_edges(src, dst):
    npad = E_PAD - E
    src_p = jnp.concatenate([src, jnp.zeros((npad,), _i32)])
    dst_p = jnp.concatenate([dst, jnp.full((npad,), V, _i32)])
    return src_p, dst_p


def kernel(x_p, e_p, edge_index_p, graph_id_p, x_l, e_l, edge_index_l,
           graph_id_l, W_node_p, W_edge_p, W_node_l, W_edge_l, W_gcn, W_gat,
           A_gat, W_d1, W_d2):
    x_p64 = jnp.pad(x_p, ((0, 0), (0, 6)))
    x_l64 = jnp.pad(x_l, ((0, 0), (0, 6)))
    wnp = jnp.pad(W_node_p, ((0, 6), (0, 0)))
    wnl = jnp.pad(W_node_l, ((0, 6), (0, 0)))
    e_p8 = jnp.pad(e_p, ((0, E_PAD - E), (0, 2)))
    e_l8 = jnp.pad(e_l, ((0, E_PAD - E), (0, 2)))
    wep = jnp.pad(W_edge_p, ((0, 2), (0, 0)))
    wel = jnp.pad(W_edge_l, ((0, 2), (0, 0)))
    src_p, dst_p = _pad_edges(edge_index_p[0], edge_index_p[1])
    src_l, dst_l = _pad_edges(edge_index_l[0], edge_index_l[1])
    gid_p = graph_id_p.astype(_f32).reshape(1, V)
    gid_l = graph_id_l.astype(_f32).reshape(1, V)
    w2pad = jnp.pad(W_d2, ((0, 0), (0, 7)))
    edge_live = (jnp.arange(E_PAD) < E)

    h_p = _mm(x_p64, wnp, 2000)
    h_l = _mm(x_l64, wnl, 2000)
    eij_p = _mm(e_p8, wep, 2048)

    hw = _mm(h_p, W_gcn[0], 2000)
    p = _gcn_edge(hw, eij_p, src_p, dst_p)
    for i in range(1, NL):
        h_p, hw = _gcn_update(h_p, p[0, :V], p[1, :V], W_gcn[i])
        p = _gcn_edge(hw, eij_p, src_p, dst_p)
    r_p = _readout(h_p, p[0, :V], p[1, :V], gid_p)

    h = h_l
    part = None
    for j in range(NL):
        a01 = jnp.pad(jnp.stack([A_gat[j, 0], A_gat[j, 1]], axis=1),
                      ((0, 0), (0, HID - 2)))
        if j == 0:
            z, s01 = _gat_pre_first(h, W_gat[0], a01)
        else:
            h, z, s01 = _gat_pre(h, part[0, :V], part[1, :V], W_gat[j], a01)
        a2col = jnp.pad(A_gat[j, 2].reshape(HID, 1), ((0, 0), (0, 7)))
        t8 = _gat_t(e_l8, wel, a2col)
        t = jnp.where(edge_live, t8[:, 0], -1e30)
        s0 = jnp.pad(s01[:, 0], (0, VACC - V))
        s1 = jnp.pad(s01[:, 1], (0, VACC - V))
        ex, dpart = _gat_scalar(s0, s1, t, src_l, dst_l)
        part = _gat_rows(z, ex, dpart, src_l, dst_l)
    r_l = _readout(h, part[0, :V], part[1, :V], gid_l)

    out = _head(r_p, r_l, W_d1, w2pad)
    return out[:, :1]


# speedup vs baseline: 4.1444x; 1.5079x over previous
import functools

import jax
import jax.numpy as jnp
from jax import lax
from jax.experimental import pallas as pl
from jax.experimental.pallas import tpu as pltpu
from jax.experimental.pallas import tpu_sc as plsc

V = 10000
E = 320000
HID = 128
NG = 64
NL = 3

NC = 2
NS = 16
NW = NC * NS
EPW = 10240
E_PAD = NW * EPW
VACC = 10240

RC = 64
RNCH = EPW // RC
SC2 = 128
SNCH = EPW // SC2

_f32 = jnp.float32
_i32 = jnp.int32

_SC_MESH = plsc.VectorSubcoreMesh(core_axis_name="c", subcore_axis_name="s")


def _mm_body(x_ref, w_ref, o_ref):
    o_ref[...] = jnp.dot(x_ref[...], w_ref[...], preferred_element_type=_f32)


def _mm(x, w, bm):
    m, k = x.shape
    n = w.shape[1]
    return pl.pallas_call(
        _mm_body,
        grid=(m // bm,),
        in_specs=[
            pl.BlockSpec((bm, k), lambda i: (i, 0)),
            pl.BlockSpec((k, n), lambda i: (0, 0)),
        ],
        out_specs=pl.BlockSpec((bm, n), lambda i: (i, 0)),
        out_shape=jax.ShapeDtypeStruct((m, n), _f32),
    )(x, w)


def _gcn_update_body(h_ref, p0_ref, p1_ref, w_ref, hn_ref, hw_ref):
    hn = jnp.maximum(h_ref[...] + p0_ref[...] + p1_ref[...], 0.0)
    hn_ref[...] = hn
    hw_ref[...] = jnp.dot(hn, w_ref[...], preferred_element_type=_f32)


def _gcn_update(h, p0, p1, w, bm=2000):
    return pl.pallas_call(
        _gcn_update_body,
        grid=(V // bm,),
        in_specs=[
            pl.BlockSpec((bm, HID), lambda i: (i, 0)),
            pl.BlockSpec((bm, HID), lambda i: (i, 0)),
            pl.BlockSpec((bm, HID), lambda i: (i, 0)),
            pl.BlockSpec((HID, HID), lambda i: (0, 0)),
        ],
        out_specs=[
            pl.BlockSpec((bm, HID), lambda i: (i, 0)),
            pl.BlockSpec((bm, HID), lambda i: (i, 0)),
        ],
        out_shape=[
            jax.ShapeDtypeStruct((V, HID), _f32),
            jax.ShapeDtypeStruct((V, HID), _f32),
        ],
    )(h, p0, p1, w)


def _gat_pre_first_body(h_ref, w_ref, a_ref, z_ref, s_ref):
    z = jnp.dot(h_ref[...], w_ref[...], preferred_element_type=_f32)
    z_ref[...] = z
    s_ref[...] = jnp.dot(z, a_ref[...], preferred_element_type=_f32)


def _gat_pre_first(h, w, a01, bm=2000):
    return pl.pallas_call(
        _gat_pre_first_body,
        grid=(V // bm,),
        in_specs=[
            pl.BlockSpec((bm, HID), lambda i: (i, 0)),
            pl.BlockSpec((HID, HID), lambda i: (0, 0)),
            pl.BlockSpec((HID, HID), lambda i: (0, 0)),
        ],
        out_specs=[
            pl.BlockSpec((bm, HID), lambda i: (i, 0)),
            pl.BlockSpec((bm, HID), lambda i: (i, 0)),
        ],
        out_shape=[
            jax.ShapeDtypeStruct((V, HID), _f32),
            jax.ShapeDtypeStruct((V, HID), _f32),
        ],
    )(h, w, a01)


def _gat_pre_body(h_ref, p0_ref, p1_ref, w_ref, a_ref, hn_ref, z_ref, s_ref):
    hn = jnp.maximum(h_ref[...] + p0_ref[...] + p1_ref[...], 0.0)
    hn_ref[...] = hn
    z = jnp.dot(hn, w_ref[...], preferred_element_type=_f32)
    z_ref[...] = z
    s_ref[...] = jnp.dot(z, a_ref[...], preferred_element_type=_f32)


def _gat_pre(h, p0, p1, w, a01, bm=2000):
    return pl.pallas_call(
        _gat_pre_body,
        grid=(V // bm,),
        in_specs=[
            pl.BlockSpec((bm, HID), lambda i: (i, 0)),
            pl.BlockSpec((bm, HID), lambda i: (i, 0)),
            pl.BlockSpec((bm, HID), lambda i: (i, 0)),
            pl.BlockSpec((HID, HID), lambda i: (0, 0)),
            pl.BlockSpec((HID, HID), lambda i: (0, 0)),
        ],
        out_specs=[
            pl.BlockSpec((bm, HID), lambda i: (i, 0)),
            pl.BlockSpec((bm, HID), lambda i: (i, 0)),
            pl.BlockSpec((bm, HID), lambda i: (i, 0)),
        ],
        out_shape=[
            jax.ShapeDtypeStruct((V, HID), _f32),
            jax.ShapeDtypeStruct((V, HID), _f32),
            jax.ShapeDtypeStruct((V, HID), _f32),
        ],
    )(h, p0, p1, w, a01)


def _gat_t_body(e_ref, we_ref, a2_ref, t_ref):
    w2 = jnp.dot(we_ref[...], a2_ref[...], preferred_element_type=_f32)
    t_ref[...] = jnp.dot(e_ref[...], w2, preferred_element_type=_f32)


def _gat_t(e_pad, we_pad, a2col, bm=2048):
    m = e_pad.shape[0]
    return pl.pallas_call(
        _gat_t_body,
        grid=(m // bm,),
        in_specs=[
            pl.BlockSpec((bm, 8), lambda i: (i, 0)),
            pl.BlockSpec((8, HID), lambda i: (0, 0)),
            pl.BlockSpec((HID, 8), lambda i: (0, 0)),
        ],
        out_specs=pl.BlockSpec((bm, 8), lambda i: (i, 0)),
        out_shape=jax.ShapeDtypeStruct((m, 8), _f32),
    )(e_pad, we_pad, a2col)


def _recip_body(d_ref, o_ref):
    s = d_ref[...]
    o_ref[...] = 1.0 / (s[0:1, :] + s[1:2, :] + 1e-9)


def _recip(d):
    return pl.pallas_call(
        _recip_body,
        out_shape=jax.ShapeDtypeStruct((1, VACC), _f32),
    )(d)


def _readout_body(h_ref, p0_ref, p1_ref, gid_ref, o_ref):
    hf = jnp.maximum(h_ref[...] + p0_ref[...] + p1_ref[...], 0.0)
    iota = lax.broadcasted_iota(_i32, (NG, V), 0).astype(_f32)
    onehot = (gid_ref[...] == iota).astype(_f32)
    o_ref[...] = jnp.dot(onehot, hf, preferred_element_type=_f32)


def _readout(h, p0, p1, gid_row):
    return pl.pallas_call(
        _readout_body,
        out_shape=jax.ShapeDtypeStruct((NG, HID), _f32),
    )(h, p0, p1, gid_row)


def _head_body(rp_ref, rl_ref, w1_ref, w2_ref, o_ref):
    s = rp_ref[...] + rl_ref[...]
    hdn = jnp.maximum(jnp.dot(s, w1_ref[...], preferred_element_type=_f32), 0.0)
    o_ref[...] = jnp.dot(hdn, w2_ref[...], preferred_element_type=_f32)


def _head(rp, rl, w1, w2pad):
    return pl.pallas_call(
        _head_body,
        out_shape=jax.ShapeDtypeStruct((NG, 8), _f32),
    )(rp, rl, w1, w2pad)


def _wait(src, dst, sem):
    pltpu.make_async_copy(src, dst, sem).wait()


def _zero_block(buf):
    def zrow(r, carry):
        for c8 in range(8):
            buf[r, pl.ds(c8 * 16, 16)] = jnp.zeros((16,), _f32)
        return carry
    lax.fori_loop(0, RC, zrow, 0)


def _init_acc(rows0, acc, sid):
    _zero_block(rows0)
    for kk in range(VACC // NS // RC):
        pltpu.sync_copy(rows0, acc.at[pl.ds(sid * (VACC // NS) + kk * RC, RC), :])


def _unpack_sd(sd_all, srcb, dstb, slot, chunk):
    for g in range(RC // 16):
        v = sd_all[pl.ds(chunk * RC + g * 16, 16)]
        s16 = pl.ds(g * 16, 16)
        srcb[slot, s16] = v & 0xFFFF
        dstb[slot, s16] = lax.shift_right_logical(v, 16)


def _gcn_edge_body(hw, eij, sd, out, sd_all, srcb, dstb, rows, eijb, acc,
                   gsem, esem, ssem):
    cid = lax.axis_index("c")
    sid = lax.axis_index("s")
    wid = sid * NC + cid
    base = wid * EPW
    _init_acc(rows.at[0], acc, sid)
    pltpu.sync_copy(sd.at[pl.ds(base, EPW)], sd_all)
    plsc.subcore_barrier()

    def issue_gather(slot):
        pltpu.async_copy(hw.at[srcb.at[slot]], rows.at[slot], gsem.at[slot])

    def issue_eij(chunk, slot):
        pltpu.async_copy(eij.at[pl.ds(base + chunk * RC, RC), :],
                         eijb.at[slot], esem.at[slot])

    _unpack_sd(sd_all, srcb, dstb, 0, 0)
    issue_eij(0, 0)
    issue_gather(0)

    def step(k, slot):
        nslot = 1 - slot
        _wait(hw.at[srcb.at[slot]], rows.at[slot], gsem.at[slot])
        _wait(eij.at[pl.ds(0, RC), :], eijb.at[slot], esem.at[slot])

        @pl.when(k > 0)
        def _():
            _wait(rows.at[nslot], acc.at[dstb.at[nslot]], ssem.at[nslot])

        nk = jnp.minimum(k + 1, RNCH - 1)
        _unpack_sd(sd_all, srcb, dstb, nslot, nk)
        issue_gather(nslot)
        issue_eij(nk, nslot)

        def row(r, rc):
            for c8 in range(8):
                s16 = pl.ds(c8 * 16, 16)
                rows[slot, r, s16] = jnp.maximum(
                    rows[slot, r, s16] + eijb[slot, r, s16], 0.0)
            return rc
        lax.fori_loop(0, RC, row, 0, unroll=2)
        pltpu.async_copy(rows.at[slot], acc.at[dstb.at[slot]],
                         ssem.at[slot], add=True)

    def pair(i, carry):
        step(2 * i, 0)
        step(2 * i + 1, 1)
        return carry
    lax.fori_loop(0, RNCH // 2, pair, 0)
    _wait(hw.at[srcb.at[0]], rows.at[0], gsem.at[0])
    _wait(eij.at[pl.ds(0, RC), :], eijb.at[0], esem.at[0])
    _wait(rows.at[1], acc.at[dstb.at[1]], ssem.at[1])
    plsc.subcore_barrier()
    for kk in range(5):
        r0 = sid * 640 + kk * 128
        pltpu.sync_copy(acc.at[pl.ds(r0, 128), :], out.at[cid, pl.ds(r0, 128), :])


@functools.partial(
    pl.kernel,
    out_type=jax.ShapeDtypeStruct((NC, VACC, HID), _f32),
    mesh=_SC_MESH,
    compiler_params=pltpu.CompilerParams(needs_layout_passes=False,
                                         has_side_effects=True),
    scratch_types=[
        pltpu.VMEM((EPW,), _i32),
        pltpu.VMEM((2, RC), _i32),
        pltpu.VMEM((2, RC), _i32),
        pltpu.VMEM((2, RC, HID), _f32),
        pltpu.VMEM((2, RC, HID), _f32),
        pltpu.VMEM_SHARED((VACC, HID), _f32),
        pltpu.SemaphoreType.DMA((2,)),
        pltpu.SemaphoreType.DMA((2,)),
        pltpu.SemaphoreType.DMA((2,)),
    ],
)
def _gcn_edge(hw, eij, sd, out, sd_all, srcb, dstb, rows, eijb, acc,
              gsem, esem, ssem):
    _gcn_edge_body(hw, eij, sd, out, sd_all, srcb, dstb, rows, eijb, acc,
                   gsem, esem, ssem)


def _gat_scalar_body(s0, s1, t, sd, ex_out, d_out,
                     s0b, s1b, tb, sd_all, ex_all, dstb, dacc, ssem):
    cid = lax.axis_index("c")
    sid = lax.axis_index("s")
    wid = sid * NC + cid
    base = wid * EPW

    def zrow(r, carry):
        ex_all[pl.ds(r * 16, 16)] = jnp.zeros((16,), _f32)
        return carry
    lax.fori_loop(0, VACC // NS // 16, zrow, 0)
    pltpu.sync_copy(ex_all.at[pl.ds(0, VACC // NS)],
                    dacc.at[pl.ds(sid * (VACC // NS), VACC // NS)])
    pltpu.sync_copy(s0, s0b)
    pltpu.sync_copy(s1, s1b)
    pltpu.sync_copy(t.at[pl.ds(base, EPW)], tb)
    pltpu.sync_copy(sd.at[pl.ds(base, EPW)], sd_all)
    plsc.subcore_barrier()

    def step(k, slot):
        nslot = 1 - slot

        @pl.when(k > 0)
        def _():
            _wait(ex_all.at[pl.ds(0, SC2)], dacc.at[dstb.at[nslot]],
                  ssem.at[nslot])

        for g in range(8):
            s16 = pl.ds(g * 16, 16)
            e16 = pl.ds(k * SC2 + g * 16, 16)
            v = sd_all[e16]
            isrc = v & 0xFFFF
            idst = lax.shift_right_logical(v, 16)
            dstb[slot, s16] = idst
            lg = (plsc.load_gather(s0b, [isrc])
                  + plsc.load_gather(s1b, [idst])
                  + tb[e16])
            lg = jnp.maximum(lg, 0.2 * lg)
            ex_all[e16] = jnp.exp(lg)
        pltpu.async_copy(ex_all.at[pl.ds(k * SC2, SC2)],
                         dacc.at[dstb.at[slot]], ssem.at[slot], add=True)

    def pair(i, carry):
        step(2 * i, 0)
        step(2 * i + 1, 1)
        return carry
    lax.fori_loop(0, SNCH // 2, pair, 0)
    _wait(ex_all.at[pl.ds(0, SC2)], dacc.at[dstb.at[1]], ssem.at[1])
    pltpu.sync_copy(ex_all, ex_out.at[pl.ds(base, EPW)])
    plsc.subcore_barrier()
    sl = pl.ds(sid * (VACC // NS), VACC // NS)
    pltpu.sync_copy(dacc.at[sl], d_out.at[cid, sl])


@functools.partial(
    pl.kernel,
    out_type=[
        jax.ShapeDtypeStruct((E_PAD,), _f32),
        jax.ShapeDtypeStruct((NC, VACC), _f32),
    ],
    mesh=_SC_MESH,
    compiler_params=pltpu.CompilerParams(needs_layout_passes=False,
                                         has_side_effects=True),
    scratch_types=[
        pltpu.VMEM((VACC,), _f32),
        pltpu.VMEM((VACC,), _f32),
        pltpu.VMEM((EPW,), _f32),
        pltpu.VMEM((EPW,), _i32),
        pltpu.VMEM((EPW,), _f32),
        pltpu.VMEM((2, SC2), _i32),
        pltpu.VMEM_SHARED((VACC,), _f32),
        pltpu.SemaphoreType.DMA((2,)),
    ],
)
def _gat_scalar(s0, s1, t, sd, ex_out, d_out,
                s0b, s1b, tb, sd_all, ex_all, dstb, dacc, ssem):
    _gat_scalar_body(s0, s1, t, sd, ex_out, d_out,
                     s0b, s1b, tb, sd_all, ex_all, dstb, dacc, ssem)


def _gat_rows_body(z, ex, dr, sd, out,
                   sd_all, ex_all, dbuf, srcb, dstb, ab, rows, acc,
                   gsem, ssem):
    cid = lax.axis_index("c")
    sid = lax.axis_index("s")
    wid = sid * NC + cid
    base = wid * EPW
    _init_acc(rows.at[0], acc, sid)
    pltpu.sync_copy(sd.at[pl.ds(base, EPW)], sd_all)
    pltpu.sync_copy(ex.at[pl.ds(base, EPW)], ex_all)
    pltpu.sync_copy(dr, dbuf)
    plsc.subcore_barrier()

    def issue_gather(slot):
        pltpu.async_copy(z.at[srcb.at[slot]], rows.at[slot], gsem.at[slot])

    _unpack_sd(sd_all, srcb, dstb, 0, 0)
    issue_gather(0)

    def step(k, slot):
        nslot = 1 - slot
        _wait(z.at[srcb.at[slot]], rows.at[slot], gsem.at[slot])

        @pl.when(k > 0)
        def _():
            _wait(rows.at[nslot], acc.at[dstb.at[nslot]], ssem.at[nslot])

        nk = jnp.minimum(k + 1, RNCH - 1)
        _unpack_sd(sd_all, srcb, dstb, nslot, nk)
        issue_gather(nslot)
        for g in range(RC // 16):
            s16 = pl.ds(g * 16, 16)
            idst = dstb[slot, s16]
            ab[s16] = (ex_all[pl.ds(k * RC + g * 16, 16)]
                       * plsc.load_gather(dbuf, [idst]))

        def row(r, rc):
            asp = plsc.load_gather(ab, [jnp.zeros((16,), _i32) + r])
            for c8 in range(8):
                s16 = pl.ds(c8 * 16, 16)
                rows[slot, r, s16] = rows[slot, r, s16] * asp
            return rc
        lax.fori_loop(0, RC, row, 0, unroll=2)
        pltpu.async_copy(rows.at[slot], acc.at[dstb.at[slot]],
                         ssem.at[slot], add=True)

    def pair(i, carry):
        step(2 * i, 0)
        step(2 * i + 1, 1)
        return carry
    lax.fori_loop(0, RNCH // 2, pair, 0)
    _wait(z.at[srcb.at[0]], rows.at[0], gsem.at[0])
    _wait(rows.at[1], acc.at[dstb.at[1]], ssem.at[1])
    plsc.subcore_barrier()
    for kk in range(5):
        r0 = sid * 640 + kk * 128
        pltpu.sync_copy(acc.at[pl.ds(r0, 128), :], out.at[cid, pl.ds(r0, 128), :])


@functools.partial(
    pl.kernel,
    out_type=jax.ShapeDtypeStruct((NC, VACC, HID), _f32),
    mesh=_SC_MESH,
    compiler_params=pltpu.CompilerParams(needs_layout_passes=False,
                                         has_side_effects=True),
    scratch_types=[
        pltpu.VMEM((EPW,), _i32),
        pltpu.VMEM((EPW,), _f32),
        pltpu.VMEM((VACC,), _f32),
        pltpu.VMEM((2, RC), _i32),
        pltpu.VMEM((2, RC), _i32),
        pltpu.VMEM((RC,), _f32),
        pltpu.VMEM((2, RC, HID), _f32),
        pltpu.VMEM_SHARED((VACC, HID), _f32),
        pltpu.SemaphoreType.DMA((2,)),
        pltpu.SemaphoreType.DMA((2,)),
    ],
)
def _gat_rows(z, ex, dr, sd, out,
              sd_all, ex_all, dbuf, srcb, dstb, ab, rows, acc, gsem, ssem):
    _gat_rows_body(z, ex, dr, sd, out,
                   sd_all, ex_all, dbuf, srcb, dstb, ab, rows, acc,
                   gsem, ssem)


def _pack_edges(src, dst):
    npad = E_PAD - E
    src_p = jnp.concatenate([src, jnp.zeros((npad,), _i32)])
    dst_p = jnp.concatenate([dst, jnp.full((npad,), V, _i32)])
    return jnp.bitwise_or(src_p, dst_p << 16)


def kernel(x_p, e_p, edge_index_p, graph_id_p, x_l, e_l, edge_index_l,
           graph_id_l, W_node_p, W_edge_p, W_node_l, W_edge_l, W_gcn, W_gat,
           A_gat, W_d1, W_d2):
    x_p64 = jnp.pad(x_p, ((0, 0), (0, 6)))
    x_l64 = jnp.pad(x_l, ((0, 0), (0, 6)))
    wnp = jnp.pad(W_node_p, ((0, 6), (0, 0)))
    wnl = jnp.pad(W_node_l, ((0, 6), (0, 0)))
    e_p8 = jnp.pad(e_p, ((0, E_PAD - E), (0, 2)))
    e_l8 = jnp.pad(e_l, ((0, E_PAD - E), (0, 2)))
    wep = jnp.pad(W_edge_p, ((0, 2), (0, 0)))
    wel = jnp.pad(W_edge_l, ((0, 2), (0, 0)))
    sd_p = _pack_edges(edge_index_p[0], edge_index_p[1])
    sd_l = _pack_edges(edge_index_l[0], edge_index_l[1])
    gid_p = graph_id_p.astype(_f32).reshape(1, V)
    gid_l = graph_id_l.astype(_f32).reshape(1, V)
    w2pad = jnp.pad(W_d2, ((0, 0), (0, 7)))
    edge_live = (jnp.arange(E_PAD) < E)

    h_p = _mm(x_p64, wnp, 2000)
    h_l = _mm(x_l64, wnl, 2000)
    eij_p = _mm(e_p8, wep, 2048)

    hw = _mm(h_p, W_gcn[0], 2000)
    p = _gcn_edge(hw, eij_p, sd_p)
    for i in range(1, NL):
        h_p, hw = _gcn_update(h_p, p[0, :V], p[1, :V], W_gcn[i])
        p = _gcn_edge(hw, eij_p, sd_p)
    r_p = _readout(h_p, p[0, :V], p[1, :V], gid_p)

    h = h_l
    part = None
    for j in range(NL):
        a01 = jnp.pad(jnp.stack([A_gat[j, 0], A_gat[j, 1]], axis=1),
                      ((0, 0), (0, HID - 2)))
        if j == 0:
            z, s01 = _gat_pre_first(h, W_gat[0], a01)
        else:
            h, z, s01 = _gat_pre(h, part[0, :V], part[1, :V], W_gat[j], a01)
        a2col = jnp.pad(A_gat[j, 2].reshape(HID, 1), ((0, 0), (0, 7)))
        t8 = _gat_t(e_l8, wel, a2col)
        t = jnp.where(edge_live, t8[:, 0], -1e30)
        s0 = jnp.pad(s01[:, 0], (0, VACC - V))
        s1 = jnp.pad(s01[:, 1], (0, VACC - V))
        ex, dpart = _gat_scalar(s0, s1, t, sd_l)
        dr = _recip(dpart).reshape(VACC)
        part = _gat_rows(z, ex, dr, sd_l)
    r_l = _readout(h, part[0, :V], part[1, :V], gid_l)

    out = _head(r_p, r_l, W_d1, w2pad)
    return out[:, :1]
